# baseline (speedup 1.0000x reference)
import functools

import jax
import jax.numpy as jnp
from jax import lax
from jax.experimental import pallas as pl
from jax.experimental.pallas import tpu as pltpu
from jax.experimental.pallas import tpu_sc as plsc

_NUM_CORES = 2
_NUM_SUBCORES = 16
_NUM_WORKERS = _NUM_CORES * _NUM_SUBCORES

_BATCH = 16384
_D_MODEL = 768
_ROWS_PER_WORKER = _BATCH // _NUM_WORKERS
_BUF_ROWS = 80
_CHUNKS = (80, 80, 80, 80, 80, 80, 32)
_OFFS = tuple(sum(_CHUNKS[:i]) for i in range(len(_CHUNKS)))


@functools.partial(
    pl.kernel,
    out_type=jax.ShapeDtypeStruct((_BATCH, _D_MODEL), jnp.float32),
    mesh=plsc.VectorSubcoreMesh(core_axis_name="c", subcore_axis_name="s"),
    scratch_types=[
        pltpu.VMEM((_ROWS_PER_WORKER,), jnp.int32),
        pltpu.VMEM((_BUF_ROWS, _D_MODEL), jnp.float32),
        pltpu.VMEM((_BUF_ROWS, _D_MODEL), jnp.float32),
        pltpu.SemaphoreType.DMA,
        pltpu.SemaphoreType.DMA,
        pltpu.SemaphoreType.DMA,
        pltpu.SemaphoreType.DMA,
    ],
)
def _sc_gather(idx_hbm, table_hbm, out_hbm, idx_v, rows0, rows1,
               g0, g1, o0, o1):
    wid = lax.axis_index("s") * _NUM_CORES + lax.axis_index("c")
    base = wid * _ROWS_PER_WORKER
    c0 = _CHUNKS[0]
    pltpu.sync_copy(idx_hbm.at[pl.ds(base, c0)], idx_v.at[pl.ds(0, c0)])

    bufs = (rows0, rows1)
    gsems = (g0, g1)
    osems = (o0, o1)
    n = len(_CHUNKS)

    def gather(k, buf, sem):
        c = _CHUNKS[k]
        return pltpu.async_copy(
            table_hbm.at[idx_v.at[pl.ds(_OFFS[k], c)]],
            buf.at[pl.ds(0, c)], sem)

    def put(k, buf, sem):
        c = _CHUNKS[k]
        return pltpu.async_copy(
            buf.at[pl.ds(0, c)], out_hbm.at[pl.ds(base + _OFFS[k], c)], sem)

    g_descs = [gather(0, bufs[0], gsems[0]), None]
    pltpu.sync_copy(idx_hbm.at[pl.ds(base + c0, _ROWS_PER_WORKER - c0)],
                    idx_v.at[pl.ds(c0, _ROWS_PER_WORKER - c0)])
    out_descs = [None, None]
    for i in range(n):
        b, nb = i % 2, (i + 1) % 2
        g_descs[b].wait()
        out_descs[b] = put(i, bufs[b], osems[b])
        if i + 1 < n:
            if out_descs[nb] is not None:
                out_descs[nb].wait()
            g_descs[nb] = gather(i + 1, bufs[nb], gsems[nb])
    out_descs[(n - 1) % 2].wait()
    out_descs[(n - 2) % 2].wait()


def kernel(x, y, W_lookup):
    del x
    return _sc_gather(y, W_lookup)


# speedup vs baseline: 1.0669x; 1.0669x over previous
import functools

import jax
import jax.numpy as jnp
from jax import lax
from jax.experimental import pallas as pl
from jax.experimental.pallas import tpu as pltpu
from jax.experimental.pallas import tpu_sc as plsc

_NUM_CORES = 2
_NUM_SUBCORES = 16
_NUM_WORKERS = _NUM_CORES * _NUM_SUBCORES

_BATCH = 16384
_D_MODEL = 768
_ROWS_PER_WORKER = _BATCH // _NUM_WORKERS
_BUF_ROWS = 56
_CHUNKS = (56,) * 9 + (8,)
_OFFS = tuple(sum(_CHUNKS[:i]) for i in range(len(_CHUNKS)))
_NBUF = 3


@functools.partial(
    pl.kernel,
    out_type=jax.ShapeDtypeStruct((_BATCH, _D_MODEL), jnp.float32),
    mesh=plsc.VectorSubcoreMesh(core_axis_name="c", subcore_axis_name="s"),
    scratch_types=[
        pltpu.VMEM((_ROWS_PER_WORKER,), jnp.int32),
        pltpu.VMEM((_BUF_ROWS, _D_MODEL), jnp.float32),
        pltpu.VMEM((_BUF_ROWS, _D_MODEL), jnp.float32),
        pltpu.VMEM((_BUF_ROWS, _D_MODEL), jnp.float32),
        pltpu.SemaphoreType.DMA,
        pltpu.SemaphoreType.DMA,
        pltpu.SemaphoreType.DMA,
        pltpu.SemaphoreType.DMA,
        pltpu.SemaphoreType.DMA,
        pltpu.SemaphoreType.DMA,
    ],
)
def _sc_gather(idx_hbm, table_hbm, out_hbm, idx_v, rows0, rows1, rows2,
               g0, g1, g2, o0, o1, o2):
    wid = lax.axis_index("s") * _NUM_CORES + lax.axis_index("c")
    base = wid * _ROWS_PER_WORKER
    head = _OFFS[2]
    pltpu.sync_copy(idx_hbm.at[pl.ds(base, head)], idx_v.at[pl.ds(0, head)])

    bufs = (rows0, rows1, rows2)
    gsems = (g0, g1, g2)
    osems = (o0, o1, o2)
    n = len(_CHUNKS)

    def gather(k, buf, sem):
        c = _CHUNKS[k]
        return pltpu.async_copy(
            table_hbm.at[idx_v.at[pl.ds(_OFFS[k], c)]],
            buf.at[pl.ds(0, c)], sem)

    def put(k, buf, sem):
        c = _CHUNKS[k]
        return pltpu.async_copy(
            buf.at[pl.ds(0, c)], out_hbm.at[pl.ds(base + _OFFS[k], c)], sem)

    g_descs = [gather(0, bufs[0], gsems[0]), gather(1, bufs[1], gsems[1]),
               None]
    pltpu.sync_copy(idx_hbm.at[pl.ds(base + head, _ROWS_PER_WORKER - head)],
                    idx_v.at[pl.ds(head, _ROWS_PER_WORKER - head)])
    out_descs = [None, None, None]
    for i in range(n):
        b = i % _NBUF
        pf = i + _NBUF - 1
        if pf < n:
            nb = pf % _NBUF
            if out_descs[nb] is not None:
                out_descs[nb].wait()
            g_descs[nb] = gather(pf, bufs[nb], gsems[nb])
        g_descs[b].wait()
        out_descs[b] = put(i, bufs[b], osems[b])
    for j in range(_NBUF):
        if out_descs[(n - 1 - j) % _NBUF] is not None:
            out_descs[(n - 1 - j) % _NBUF].wait()
            out_descs[(n - 1 - j) % _NBUF] = None


def kernel(x, y, W_lookup):
    del x
    return _sc_gather(y, W_lookup)
